# Initial kernel scaffold; baseline (speedup 1.0000x reference)
import jax
import jax.numpy as jnp
from jax.experimental import pallas as pl


def kernel(mouse_cat, scroll, buttons, keys, yaw_pitch, gui, hotbar, mouse_table, scroll_table, hotbar_table, slot_table, buttons_W, buttons_b, keys_W, keys_b, yawgui_W, yawgui_b):
    raise NotImplementedError("write your pallas kernel here")


# speedup vs baseline: 5.4145x; 5.4145x over previous
import jax
import jax.numpy as jnp
from jax.experimental import pallas as pl
from jax.experimental.pallas import tpu as pltpu

_TILE = 512


def _tok_kernel(idx_ref, btn_ref, keys_ref, yg_ref,
                mouse_ref, scroll_ref, hotbar_ref,
                bW_ref, kW_ref, ygW_ref, bias_ref, out_ref):
    idx = idx_ref[...]
    f32 = jnp.float32

    m = idx[:, 0:1]
    oh0 = (m == jax.lax.broadcasted_iota(jnp.int32, (1, 121), 1)).astype(f32)
    tok0 = jnp.dot(oh0, mouse_ref[...], preferred_element_type=f32)
    tok0 = tok0 + bias_ref[0:1, :]

    s = idx[:, 1:2]
    oh1 = (s == jax.lax.broadcasted_iota(jnp.int32, (1, 3), 1)).astype(f32)
    tok1 = jnp.dot(oh1, scroll_ref[...], preferred_element_type=f32)
    tok1 = tok1 + jnp.dot(btn_ref[...], bW_ref[...], preferred_element_type=f32)
    tok1 = tok1 + bias_ref[1:2, :]

    tok2 = jnp.dot(keys_ref[...], kW_ref[...], preferred_element_type=f32)
    tok2 = tok2 + bias_ref[2:3, :]

    h = idx[:, 2:3]
    oh3 = (h == jax.lax.broadcasted_iota(jnp.int32, (1, 9), 1)).astype(f32)
    tok3 = jnp.dot(oh3, hotbar_ref[...], preferred_element_type=f32)
    tok3 = tok3 + jnp.dot(yg_ref[...], ygW_ref[...], preferred_element_type=f32)
    tok3 = tok3 + bias_ref[3:4, :]

    out_ref[:, 0, :] = tok0
    out_ref[:, 1, :] = tok1
    out_ref[:, 2, :] = tok2
    out_ref[:, 3, :] = tok3


def kernel(mouse_cat, scroll, buttons, keys, yaw_pitch, gui, hotbar,
           mouse_table, scroll_table, hotbar_table, slot_table,
           buttons_W, buttons_b, keys_W, keys_b, yawgui_W, yawgui_b):
    B, T = mouse_cat.shape
    D = mouse_table.shape[1]
    N = B * T

    idx = jnp.stack([mouse_cat, scroll, hotbar], axis=-1).reshape(N, 3)
    idx = idx.astype(jnp.int32)
    btn = buttons.reshape(N, 3)
    ky = keys.reshape(N, keys.shape[-1])
    yg = jnp.concatenate([yaw_pitch, gui], axis=-1).reshape(N, 4)

    zeros_b = jnp.zeros_like(buttons_b)
    bias = slot_table + jnp.stack([zeros_b, buttons_b, keys_b, yawgui_b], axis=0)

    grid = (N // _TILE,)

    def tok_map(i):
        return (i, 0)

    def full_map(i):
        return (0, 0)

    out = pl.pallas_call(
        _tok_kernel,
        grid=grid,
        in_specs=[
            pl.BlockSpec((_TILE, 3), tok_map),
            pl.BlockSpec((_TILE, 3), tok_map),
            pl.BlockSpec((_TILE, ky.shape[1]), tok_map),
            pl.BlockSpec((_TILE, 4), tok_map),
            pl.BlockSpec(mouse_table.shape, full_map),
            pl.BlockSpec(scroll_table.shape, full_map),
            pl.BlockSpec(hotbar_table.shape, full_map),
            pl.BlockSpec(buttons_W.shape, full_map),
            pl.BlockSpec(keys_W.shape, full_map),
            pl.BlockSpec(yawgui_W.shape, full_map),
            pl.BlockSpec(bias.shape, full_map),
        ],
        out_specs=pl.BlockSpec((_TILE, 4, D), lambda i: (i, 0, 0)),
        out_shape=jax.ShapeDtypeStruct((N, 4, D), jnp.float32),
        compiler_params=pltpu.CompilerParams(
            dimension_semantics=("arbitrary",),
        ),
    )(idx, btn, ky, yg, mouse_table, scroll_table, hotbar_table,
      buttons_W, keys_W, yawgui_W, bias)

    return out.reshape(B, T, 4, D)
